# Initial kernel scaffold; baseline (speedup 1.0000x reference)
import jax
import jax.numpy as jnp
from jax.experimental import pallas as pl


def kernel(x, edge_index, W, b):
    raise NotImplementedError("write your pallas kernel here")


# speedup vs baseline: 18.9395x; 18.9395x over previous
import math

import jax
import jax.numpy as jnp
from jax import lax
from jax.experimental import pallas as pl
from jax.experimental.pallas import tpu as pltpu
from jax.experimental.pallas import tpu_sc as plsc

N2 = 10240
CHUNK = 128
NC, NS = 2, 16
NW = NC * NS
ROWS = N2 // NS

_MESH = plsc.VectorSubcoreMesh(core_axis_name="c", subcore_axis_name="s")


def _hist_body(col_hbm, zrows_hbm, ones_hbm, out_hbm, colv, onesv, acc, sem):
    del sem
    c = lax.axis_index("c")
    s = lax.axis_index("s")
    wid = c * NS + s
    nch = col_hbm.shape[1]
    pltpu.sync_copy(zrows_hbm, acc.at[pl.ds(s * ROWS, ROWS)])
    pltpu.sync_copy(ones_hbm, onesv)
    pltpu.sync_copy(col_hbm.at[wid], colv)
    plsc.subcore_barrier()

    def step(j, carry):
        pltpu.sync_copy(onesv, acc.at[colv.at[j]], add=True)
        return carry

    lax.fori_loop(0, nch, step, 0)
    plsc.subcore_barrier()
    pltpu.sync_copy(acc.at[pl.ds(s * ROWS, ROWS)],
                    out_hbm.at[c, pl.ds(s * ROWS, ROWS)])


def _scatter_body(row_hbm, col_hbm, y_hbm, ztile_hbm, out_hbm,
                  rowv, colv, buf, acc, gsem):
    c = lax.axis_index("c")
    s = lax.axis_index("s")
    wid = c * NS + s
    nch = row_hbm.shape[1]
    pltpu.sync_copy(ztile_hbm, acc.at[pl.ds(s * ROWS, ROWS)])
    pltpu.sync_copy(row_hbm.at[wid], rowv)
    pltpu.sync_copy(col_hbm.at[wid], colv)
    plsc.subcore_barrier()

    def step(j, carry):
        pltpu.async_copy(y_hbm.at[rowv.at[j]], buf, gsem).wait()
        pltpu.sync_copy(buf, acc.at[colv.at[j]], add=True)
        return carry

    lax.fori_loop(0, nch, step, 0)
    plsc.subcore_barrier()
    pltpu.sync_copy(acc.at[pl.ds(s * ROWS, ROWS)],
                    out_hbm.at[c, pl.ds(s * ROWS, ROWS)])


def _scale_body(x_ref, w_ref, h0_ref, h1_ref, y_ref):
    deg = h0_ref[...] + h1_ref[...] + 1.0
    y_ref[...] = jnp.dot(x_ref[...], w_ref[...],
                         preferred_element_type=jnp.float32) * lax.rsqrt(deg)


def _final_body(p0_ref, p1_ref, y_ref, h0_ref, h1_ref, b_ref, o_ref):
    deg = h0_ref[...] + h1_ref[...] + 1.0
    t = (p0_ref[...] + p1_ref[...] + y_ref[...]) * lax.rsqrt(deg) + b_ref[...]
    o_ref[...] = t * 0.5 * (1.0 + lax.erf(t * (1.0 / math.sqrt(2.0))))


def kernel(x, edge_index, W, b):
    n, d = x.shape
    e = edge_index.shape[1]
    row = edge_index[0].astype(jnp.int32)
    col = edge_index[1].astype(jnp.int32)

    step = NW * CHUNK
    e_pad = step * ((e + step - 1) // step)
    nch = e_pad // step
    row3 = jnp.full((e_pad,), n, jnp.int32).at[:e].set(row).reshape(NW, nch, CHUNK)
    col3 = jnp.full((e_pad,), n, jnp.int32).at[:e].set(col).reshape(NW, nch, CHUNK)
    xpad = jnp.zeros((N2, d), jnp.float32).at[:n, :].set(x.astype(jnp.float32))

    zrows = jnp.zeros((ROWS,), jnp.float32)
    ones = jnp.ones((CHUNK,), jnp.float32)
    ztile = jnp.zeros((ROWS, d), jnp.float32)

    hist = pl.kernel(
        _hist_body,
        out_type=jax.ShapeDtypeStruct((NC, N2), jnp.float32),
        mesh=_MESH,
        scratch_types=[
            pltpu.VMEM((nch, CHUNK), jnp.int32),
            pltpu.VMEM((CHUNK,), jnp.float32),
            pltpu.VMEM_SHARED((N2,), jnp.float32),
            pltpu.SemaphoreType.DMA,
        ],
    )(col3, zrows, ones)

    h0 = hist[0].reshape(N2, 1)
    h1 = hist[1].reshape(N2, 1)

    blk = 1024
    y = pl.pallas_call(
        _scale_body,
        grid=(N2 // blk,),
        in_specs=[
            pl.BlockSpec((blk, d), lambda i: (i, 0)),
            pl.BlockSpec((d, d), lambda i: (0, 0)),
            pl.BlockSpec((blk, 1), lambda i: (i, 0)),
            pl.BlockSpec((blk, 1), lambda i: (i, 0)),
        ],
        out_specs=pl.BlockSpec((blk, d), lambda i: (i, 0)),
        out_shape=jax.ShapeDtypeStruct((N2, d), jnp.float32),
    )(xpad, W.astype(jnp.float32), h0, h1)

    parts = pl.kernel(
        _scatter_body,
        out_type=jax.ShapeDtypeStruct((NC, N2, d), jnp.float32),
        mesh=_MESH,
        scratch_types=[
            pltpu.VMEM((nch, CHUNK), jnp.int32),
            pltpu.VMEM((nch, CHUNK), jnp.int32),
            pltpu.VMEM((CHUNK, d), jnp.float32),
            pltpu.VMEM_SHARED((N2, d), jnp.float32),
            pltpu.SemaphoreType.DMA,
        ],
    )(row3, col3, y, ztile)

    out = pl.pallas_call(
        _final_body,
        grid=(N2 // blk,),
        in_specs=[
            pl.BlockSpec((blk, d), lambda i: (i, 0)),
            pl.BlockSpec((blk, d), lambda i: (i, 0)),
            pl.BlockSpec((blk, d), lambda i: (i, 0)),
            pl.BlockSpec((blk, 1), lambda i: (i, 0)),
            pl.BlockSpec((blk, 1), lambda i: (i, 0)),
            pl.BlockSpec((1, d), lambda i: (0, 0)),
        ],
        out_specs=pl.BlockSpec((blk, d), lambda i: (i, 0)),
        out_shape=jax.ShapeDtypeStruct((N2, d), jnp.float32),
    )(parts[0], parts[1], y, h0, h1, b.reshape(1, d).astype(jnp.float32))

    return out[:n]
